# baseline (speedup 1.0000x reference)
import functools

import jax
import jax.numpy as jnp
from jax import lax
from jax.experimental import pallas as pl
from jax.experimental.pallas import tpu as pltpu
from jax.experimental.pallas import tpu_sc as plsc

_N_TARGET = 64.0
_NX = 4096
_LANES = 16
_CHUNKS = _NX // _LANES
_K_RTSAFE = 7


def _butterfly(v, op):
    lanes = lax.iota(jnp.int32, _LANES)
    for s in (8, 4, 2, 1):
        v = op(v, v.at[lanes ^ s].get(mode="promise_in_bounds"))
    return v


def _lml_body(x_hbm, y_hbm, nu_hbm, x_v, y_v, nu_v, nu_shared, nu_gather):
    cid = lax.axis_index("c")
    sid = lax.axis_index("s")
    wid = cid * _LANES + sid
    pltpu.sync_copy(x_hbm.at[wid], x_v)

    def prep_step(i, carry):
        mn, mx = carry
        xn = 0.0 - x_v[pl.ds(i * _LANES, _LANES)]
        x_v[pl.ds(i * _LANES, _LANES)] = xn
        return jnp.minimum(mn, xn), jnp.maximum(mx, xn)

    big = jnp.full((_LANES,), 3.0e38, jnp.float32)
    mn, mx = plsc.parallel_loop(0, _CHUNKS, 1, unroll=8, carry=(big, -big))(prep_step)
    xl = _butterfly(mn, jnp.minimum) - 7.0
    xh = _butterfly(mx, jnp.maximum) + 7.0

    rts = 0.5 * (xl + xh)
    dx = xh - xl
    state0 = (xl, xh, rts, dx, dx, rts, jnp.full((_LANES,), jnp.inf, jnp.float32))

    def rtsafe_step(_, carry):
        xl, xh, rts, dx, dxold, best, fbest = carry
        nrts = 0.0 - rts

        def acc_step(i, carry):
            f0, q0, f1, q1 = carry
            s0 = 1.0 / (1.0 + jnp.exp(x_v[pl.ds(i * 2 * _LANES, _LANES)] + nrts))
            s1 = 1.0 / (1.0 + jnp.exp(x_v[pl.ds(i * 2 * _LANES + _LANES, _LANES)] + nrts))
            return f0 + s0, q0 + s0 * s0, f1 + s1, q1 + s1 * s1

        zero = jnp.zeros((_LANES,), jnp.float32)
        f0, q0, f1, q1 = plsc.parallel_loop(
            0, _CHUNKS // 2, 1, unroll=8, carry=(zero, zero, zero, zero)
        )(acc_step)
        fraw = _butterfly(f0 + f1, jnp.add)
        f = fraw - _N_TARGET
        df = fraw - _butterfly(q0 + q1, jnp.add) + 1e-30
        absf = jnp.abs(f)
        upd = absf < fbest
        best = jnp.where(upd, rts, best)
        fbest = jnp.where(upd, absf, fbest)
        below = f < 0.0
        xl = jnp.where(below, rts, xl)
        xh = jnp.where(below, xh, rts)
        outside = (((rts - xh) * df - f) * ((rts - xl) * df - f)) > 0.0
        slow = 2.0 * absf > jnp.abs(dxold * df)
        bisect = outside | slow
        step = f / df
        half = 0.5 * (xh - xl)
        dxold = dx
        dx = jnp.where(bisect, half, step)
        rts = jnp.where(bisect, xl + half, rts - step)
        return xl, xh, rts, dx, dxold, best, fbest

    nu = lax.fori_loop(0, _K_RTSAFE, rtsafe_step, state0)[5]
    nnu = 0.0 - nu

    nu_v[...] = nu
    pltpu.sync_copy(nu_v, nu_shared.at[pl.ds(sid * _LANES, _LANES)])

    @plsc.parallel_loop(0, _CHUNKS, 1, unroll=8)
    def y_step(i):
        xn = x_v[pl.ds(i * _LANES, _LANES)]
        y_v[pl.ds(i * _LANES, _LANES)] = 1.0 / (1.0 + jnp.exp(xn + nnu))

    pltpu.sync_copy(y_v, y_hbm.at[wid])

    plsc.subcore_barrier()

    @pl.when(sid == 0)
    def _():
        pltpu.sync_copy(nu_shared, nu_gather)
        lanes = lax.iota(jnp.int32, _LANES)
        diag = jnp.zeros((_LANES,), jnp.float32)
        for r in range(_LANES):
            row = nu_gather[pl.ds(r * _LANES, _LANES)]
            diag = jnp.where(lanes == r, row, diag)
        nu_v[...] = diag
        pltpu.sync_copy(nu_v, nu_hbm.at[pl.ds(cid * _LANES, _LANES)])


@jax.jit
def _lml_sc(x):
    y, nu = pl.kernel(
        _lml_body,
        out_type=[
            jax.ShapeDtypeStruct((32, _NX), jnp.float32),
            jax.ShapeDtypeStruct((32,), jnp.float32),
        ],
        mesh=plsc.VectorSubcoreMesh(core_axis_name="c", subcore_axis_name="s"),
        scratch_types=[
            pltpu.VMEM((_NX,), jnp.float32),
            pltpu.VMEM((_NX,), jnp.float32),
            pltpu.VMEM((_LANES,), jnp.float32),
            pltpu.VMEM_SHARED((_LANES * _LANES,), jnp.float32),
            pltpu.VMEM((_LANES * _LANES,), jnp.float32),
        ],
    )(x)
    return y, nu


def kernel(x):
    return _lml_sc(x)


# speedup vs baseline: 1.0069x; 1.0069x over previous
import functools

import jax
import jax.numpy as jnp
from jax import lax
from jax.experimental import pallas as pl
from jax.experimental.pallas import tpu as pltpu
from jax.experimental.pallas import tpu_sc as plsc

_N_TARGET = 64.0
_NX = 4096
_LANES = 16
_CHUNKS = _NX // _LANES
_K_RTSAFE = 7


def _butterfly(v, op):
    lanes = lax.iota(jnp.int32, _LANES)
    for s in (8, 4, 2, 1):
        v = op(v, v.at[lanes ^ s].get(mode="promise_in_bounds"))
    return v


def _lml_body(x_hbm, y_hbm, nu_hbm, x_v, y_v, nu_v, nu_shared, nu_gather):
    cid = lax.axis_index("c")
    sid = lax.axis_index("s")
    wid = cid * _LANES + sid
    pltpu.sync_copy(x_hbm.at[wid], x_v)

    def prep_step(i, carry):
        mn, mx = carry
        xn = 0.0 - x_v[pl.ds(i * _LANES, _LANES)]
        x_v[pl.ds(i * _LANES, _LANES)] = xn
        return jnp.minimum(mn, xn), jnp.maximum(mx, xn)

    big = jnp.full((_LANES,), 3.0e38, jnp.float32)
    mn, mx = plsc.parallel_loop(0, _CHUNKS, 1, unroll=8, carry=(big, -big))(prep_step)
    xl = _butterfly(mn, jnp.minimum) - 7.0
    xh = _butterfly(mx, jnp.maximum) + 7.0

    rts = 0.5 * (xl + xh)
    dx = xh - xl
    state0 = (xl, xh, rts, dx, dx, rts, jnp.full((_LANES,), jnp.inf, jnp.float32))

    def rtsafe_step(_, carry):
        xl, xh, rts, dx, dxold, best, fbest = carry
        nrts = 0.0 - rts

        def acc_step(i, carry):
            f0, q0, f1, q1 = carry
            s0 = 1.0 / (1.0 + jnp.exp(x_v[pl.ds(i * 2 * _LANES, _LANES)] + nrts))
            s1 = 1.0 / (1.0 + jnp.exp(x_v[pl.ds(i * 2 * _LANES + _LANES, _LANES)] + nrts))
            return f0 + s0, q0 + s0 * s0, f1 + s1, q1 + s1 * s1

        zero = jnp.zeros((_LANES,), jnp.float32)
        f0, q0, f1, q1 = lax.fori_loop(
            0, _CHUNKS // 2, acc_step, (zero, zero, zero, zero), unroll=8
        )
        fraw = _butterfly(f0 + f1, jnp.add)
        f = fraw - _N_TARGET
        df = fraw - _butterfly(q0 + q1, jnp.add) + 1e-30
        absf = jnp.abs(f)
        upd = absf < fbest
        best = jnp.where(upd, rts, best)
        fbest = jnp.where(upd, absf, fbest)
        below = f < 0.0
        xl = jnp.where(below, rts, xl)
        xh = jnp.where(below, xh, rts)
        outside = (((rts - xh) * df - f) * ((rts - xl) * df - f)) > 0.0
        slow = 2.0 * absf > jnp.abs(dxold * df)
        bisect = outside | slow
        step = f / df
        half = 0.5 * (xh - xl)
        dxold = dx
        dx = jnp.where(bisect, half, step)
        rts = jnp.where(bisect, xl + half, rts - step)
        return xl, xh, rts, dx, dxold, best, fbest

    nu = lax.fori_loop(0, _K_RTSAFE, rtsafe_step, state0)[5]
    nnu = 0.0 - nu

    nu_v[...] = nu
    pltpu.sync_copy(nu_v, nu_shared.at[pl.ds(sid * _LANES, _LANES)])

    @plsc.parallel_loop(0, _CHUNKS, 1, unroll=8)
    def y_step(i):
        xn = x_v[pl.ds(i * _LANES, _LANES)]
        y_v[pl.ds(i * _LANES, _LANES)] = 1.0 / (1.0 + jnp.exp(xn + nnu))

    pltpu.sync_copy(y_v, y_hbm.at[wid])

    plsc.subcore_barrier()

    @pl.when(sid == 0)
    def _():
        pltpu.sync_copy(nu_shared, nu_gather)
        lanes = lax.iota(jnp.int32, _LANES)
        diag = jnp.zeros((_LANES,), jnp.float32)
        for r in range(_LANES):
            row = nu_gather[pl.ds(r * _LANES, _LANES)]
            diag = jnp.where(lanes == r, row, diag)
        nu_v[...] = diag
        pltpu.sync_copy(nu_v, nu_hbm.at[pl.ds(cid * _LANES, _LANES)])


@jax.jit
def _lml_sc(x):
    y, nu = pl.kernel(
        _lml_body,
        out_type=[
            jax.ShapeDtypeStruct((32, _NX), jnp.float32),
            jax.ShapeDtypeStruct((32,), jnp.float32),
        ],
        mesh=plsc.VectorSubcoreMesh(core_axis_name="c", subcore_axis_name="s"),
        scratch_types=[
            pltpu.VMEM((_NX,), jnp.float32),
            pltpu.VMEM((_NX,), jnp.float32),
            pltpu.VMEM((_LANES,), jnp.float32),
            pltpu.VMEM_SHARED((_LANES * _LANES,), jnp.float32),
            pltpu.VMEM((_LANES * _LANES,), jnp.float32),
        ],
    )(x)
    return y, nu


def kernel(x):
    return _lml_sc(x)


# speedup vs baseline: 1.0531x; 1.0459x over previous
import functools

import jax
import jax.numpy as jnp
from jax import lax
from jax.experimental import pallas as pl
from jax.experimental.pallas import tpu as pltpu
from jax.experimental.pallas import tpu_sc as plsc

_N_TARGET = 64.0
_NX = 4096
_LANES = 16
_CHUNKS = _NX // _LANES
_K_RTSAFE = 7


def _butterfly(v, op):
    lanes = lax.iota(jnp.int32, _LANES)
    for s in (8, 4, 2, 1):
        v = op(v, v.at[lanes ^ s].get(mode="promise_in_bounds"))
    return v


def _lml_body(x_hbm, y_hbm, nu_hbm, x_v, y_v, nu_v, nu_shared, nu_gather):
    cid = lax.axis_index("c")
    sid = lax.axis_index("s")
    wid = cid * _LANES + sid
    pltpu.sync_copy(x_hbm.at[wid], x_v)

    def prep_step(i, carry):
        mn, mx = carry
        xn = 0.0 - x_v[pl.ds(i * _LANES, _LANES)]
        x_v[pl.ds(i * _LANES, _LANES)] = xn
        return jnp.minimum(mn, xn), jnp.maximum(mx, xn)

    big = jnp.full((_LANES,), 3.0e38, jnp.float32)
    mn, mx = lax.fori_loop(0, _CHUNKS, prep_step, (big, -big), unroll=8)
    xl = _butterfly(mn, jnp.minimum) - 7.0
    xh = _butterfly(mx, jnp.maximum) + 7.0

    rts = 0.5 * (xl + xh)
    dx = xh - xl
    state0 = (xl, xh, rts, dx, dx, rts, jnp.full((_LANES,), jnp.inf, jnp.float32))

    def rtsafe_step(_, carry):
        xl, xh, rts, dx, dxold, best, fbest = carry
        nrts = 0.0 - rts

        def acc_step(i, carry):
            f0, q0, f1, q1 = carry
            s0 = 1.0 / (1.0 + jnp.exp(x_v[pl.ds(i * 2 * _LANES, _LANES)] + nrts))
            s1 = 1.0 / (1.0 + jnp.exp(x_v[pl.ds(i * 2 * _LANES + _LANES, _LANES)] + nrts))
            return f0 + s0, q0 + s0 * s0, f1 + s1, q1 + s1 * s1

        zero = jnp.zeros((_LANES,), jnp.float32)
        f0, q0, f1, q1 = lax.fori_loop(
            0, _CHUNKS // 2, acc_step, (zero, zero, zero, zero), unroll=8
        )
        fraw = _butterfly(f0 + f1, jnp.add)
        f = fraw - _N_TARGET
        df = fraw - _butterfly(q0 + q1, jnp.add) + 1e-30
        absf = jnp.abs(f)
        upd = absf < fbest
        best = jnp.where(upd, rts, best)
        fbest = jnp.where(upd, absf, fbest)
        below = f < 0.0
        xl = jnp.where(below, rts, xl)
        xh = jnp.where(below, xh, rts)
        outside = (((rts - xh) * df - f) * ((rts - xl) * df - f)) > 0.0
        slow = 2.0 * absf > jnp.abs(dxold * df)
        bisect = outside | slow
        step = f / df
        half = 0.5 * (xh - xl)
        dxold = dx
        dx = jnp.where(bisect, half, step)
        rts = jnp.where(bisect, xl + half, rts - step)
        return xl, xh, rts, dx, dxold, best, fbest

    nu = lax.fori_loop(0, _K_RTSAFE, rtsafe_step, state0)[5]
    nnu = 0.0 - nu

    nu_v[...] = nu
    pltpu.sync_copy(nu_v, nu_shared.at[pl.ds(sid * _LANES, _LANES)])

    @plsc.parallel_loop(0, _CHUNKS, 1, unroll=8)
    def y_step(i):
        xn = x_v[pl.ds(i * _LANES, _LANES)]
        y_v[pl.ds(i * _LANES, _LANES)] = 1.0 / (1.0 + jnp.exp(xn + nnu))

    pltpu.sync_copy(y_v, y_hbm.at[wid])

    plsc.subcore_barrier()

    @pl.when(sid == 0)
    def _():
        pltpu.sync_copy(nu_shared, nu_gather)
        lanes = lax.iota(jnp.int32, _LANES)
        diag = jnp.zeros((_LANES,), jnp.float32)
        for r in range(_LANES):
            row = nu_gather[pl.ds(r * _LANES, _LANES)]
            diag = jnp.where(lanes == r, row, diag)
        nu_v[...] = diag
        pltpu.sync_copy(nu_v, nu_hbm.at[pl.ds(cid * _LANES, _LANES)])


@jax.jit
def _lml_sc(x):
    y, nu = pl.kernel(
        _lml_body,
        out_type=[
            jax.ShapeDtypeStruct((32, _NX), jnp.float32),
            jax.ShapeDtypeStruct((32,), jnp.float32),
        ],
        mesh=plsc.VectorSubcoreMesh(core_axis_name="c", subcore_axis_name="s"),
        scratch_types=[
            pltpu.VMEM((_NX,), jnp.float32),
            pltpu.VMEM((_NX,), jnp.float32),
            pltpu.VMEM((_LANES,), jnp.float32),
            pltpu.VMEM_SHARED((_LANES * _LANES,), jnp.float32),
            pltpu.VMEM((_LANES * _LANES,), jnp.float32),
        ],
    )(x)
    return y, nu


def kernel(x):
    return _lml_sc(x)


# speedup vs baseline: 1.0583x; 1.0049x over previous
import jax
import jax.numpy as jnp
from jax import lax
from jax.experimental import pallas as pl
from jax.experimental.pallas import tpu as pltpu
from jax.experimental.pallas import tpu_sc as plsc

_N_TARGET = 64.0
_NX = 4096
_LANES = 16
_CHUNKS = _NX // _LANES
_K_RTSAFE = 7


def _butterfly(v, op):
    lanes = lax.iota(jnp.int32, _LANES)
    for s in (8, 4, 2, 1):
        v = op(v, v.at[lanes ^ s].get(mode="promise_in_bounds"))
    return v


def _lml_body(x_hbm, y_hbm, nu_hbm, x_v, y_v, nu_v, nu_shared, nu_gather):
    cid = lax.axis_index("c")
    sid = lax.axis_index("s")
    wid = cid * _LANES + sid
    pltpu.sync_copy(x_hbm.at[wid], x_v)

    def prep_step(i, carry):
        mn, mx = carry
        xn = 0.0 - x_v[pl.ds(i * _LANES, _LANES)]
        x_v[pl.ds(i * _LANES, _LANES)] = xn
        return jnp.minimum(mn, xn), jnp.maximum(mx, xn)

    big = jnp.full((_LANES,), 3.0e38, jnp.float32)
    mn, mx = lax.fori_loop(0, _CHUNKS, prep_step, (big, -big), unroll=8)
    xl = _butterfly(mn, jnp.minimum) - 7.0
    xh = _butterfly(mx, jnp.maximum) + 7.0

    rts = 0.5 * (xl + xh)
    dx = xh - xl
    state0 = (xl, xh, rts, dx, dx, rts, jnp.full((_LANES,), jnp.inf, jnp.float32))

    def rtsafe_step(_, carry):
        xl, xh, rts, dx, dxold, best, fbest = carry
        nrts = 0.0 - rts

        def acc_step(i, carry):
            f0, q0, f1, q1 = carry
            s0 = 1.0 / (1.0 + jnp.exp(x_v[pl.ds(i * 2 * _LANES, _LANES)] + nrts))
            s1 = 1.0 / (1.0 + jnp.exp(x_v[pl.ds(i * 2 * _LANES + _LANES, _LANES)] + nrts))
            return f0 + s0, q0 + s0 * s0, f1 + s1, q1 + s1 * s1

        zero = jnp.zeros((_LANES,), jnp.float32)
        f0, q0, f1, q1 = lax.fori_loop(
            0, _CHUNKS // 2, acc_step, (zero, zero, zero, zero), unroll=8
        )
        fraw = _butterfly(f0 + f1, jnp.add)
        f = fraw - _N_TARGET
        df = fraw - _butterfly(q0 + q1, jnp.add) + 1e-30
        absf = jnp.abs(f)
        upd = absf < fbest
        best = jnp.where(upd, rts, best)
        fbest = jnp.where(upd, absf, fbest)
        below = f < 0.0
        xl = jnp.where(below, rts, xl)
        xh = jnp.where(below, xh, rts)
        outside = (((rts - xh) * df - f) * ((rts - xl) * df - f)) > 0.0
        slow = 2.0 * absf > jnp.abs(dxold * df)
        bisect = outside | slow
        step = f / df
        half = 0.5 * (xh - xl)
        dxold = dx
        dx = jnp.where(bisect, half, step)
        rts = jnp.where(bisect, xl + half, rts - step)
        return xl, xh, rts, dx, dxold, best, fbest

    nu = lax.fori_loop(0, _K_RTSAFE, rtsafe_step, state0)[5]
    nnu = 0.0 - nu

    nu_v[...] = nu
    pltpu.sync_copy(nu_v, nu_shared.at[pl.ds(sid * _LANES, _LANES)])

    @plsc.parallel_loop(0, _CHUNKS, 1, unroll=8)
    def y_step(i):
        xn = x_v[pl.ds(i * _LANES, _LANES)]
        y_v[pl.ds(i * _LANES, _LANES)] = 1.0 / (1.0 + jnp.exp(xn + nnu))

    pltpu.sync_copy(y_v, y_hbm.at[wid])

    plsc.subcore_barrier()

    @pl.when(sid == 0)
    def _():
        pltpu.sync_copy(nu_shared, nu_gather)
        lanes = lax.iota(jnp.int32, _LANES)
        diag = jnp.zeros((_LANES,), jnp.float32)
        for r in range(_LANES):
            row = nu_gather[pl.ds(r * _LANES, _LANES)]
            diag = jnp.where(lanes == r, row, diag)
        nu_v[...] = diag
        pltpu.sync_copy(nu_v, nu_hbm.at[pl.ds(cid * _LANES, _LANES)])


@jax.jit
def _lml_sc(x):
    y, nu = pl.kernel(
        _lml_body,
        out_type=[
            jax.ShapeDtypeStruct((32, _NX), jnp.float32),
            jax.ShapeDtypeStruct((32,), jnp.float32),
        ],
        mesh=plsc.VectorSubcoreMesh(core_axis_name="c", subcore_axis_name="s"),
        scratch_types=[
            pltpu.VMEM((_NX,), jnp.float32),
            pltpu.VMEM((_NX,), jnp.float32),
            pltpu.VMEM((_LANES,), jnp.float32),
            pltpu.VMEM_SHARED((_LANES * _LANES,), jnp.float32),
            pltpu.VMEM((_LANES * _LANES,), jnp.float32),
        ],
    )(x)
    return y, nu


def kernel(x):
    return _lml_sc(x)
